# baseline (speedup 1.0000x reference)
import numpy as np
import jax
import jax.numpy as jnp
from jax.experimental import pallas as pl
from jax.experimental.pallas import tpu as pltpu

_NUM_SAMPLES = (16, 32)
_TEMPORAL_DIM = 1


def _subsample_indices(T, t):
    w = np.arange(t - 1, dtype=np.float32) / np.float32(t - 1)
    body = np.float32(0.0) * (np.float32(1.0) - w) + np.float32(T - 1) * w
    vals = np.concatenate([body, np.asarray([T - 1], np.float32)])
    vals = np.clip(vals, 0, T - 1)
    return vals.astype(np.int32)


def kernel(x):
    B, T, C, H, W = x.shape
    t16, t32 = _NUM_SAMPLES
    idx16 = [int(v) for v in _subsample_indices(T, t16)]
    idx32 = [int(v) for v in _subsample_indices(T, t32)]

    def body(x_ref, o16_ref, o32_ref, sem):
        copies = []
        for b in range(B):
            for j, t in enumerate(idx16):
                copies.append(pltpu.make_async_copy(
                    x_ref.at[b, t], o16_ref.at[b, j], sem))
            for j, t in enumerate(idx32):
                copies.append(pltpu.make_async_copy(
                    x_ref.at[b, t], o32_ref.at[b, j], sem))
        for c in copies:
            c.start()
        for c in copies:
            c.wait()

    return pl.pallas_call(
        body,
        in_specs=[pl.BlockSpec(memory_space=pl.ANY)],
        out_specs=[pl.BlockSpec(memory_space=pl.ANY),
                   pl.BlockSpec(memory_space=pl.ANY)],
        out_shape=[jax.ShapeDtypeStruct((B, t16, C, H, W), x.dtype),
                   jax.ShapeDtypeStruct((B, t32, C, H, W), x.dtype)],
        scratch_shapes=[pltpu.SemaphoreType.DMA],
    )(x)


# speedup vs baseline: 40.7866x; 40.7866x over previous
import functools

import numpy as np
import jax
import jax.numpy as jnp
from jax import lax
from jax.experimental import pallas as pl
from jax.experimental.pallas import tpu as pltpu
from jax.experimental.pallas import tpu_sc as plsc

_NUM_SAMPLES = (16, 32)
_TEMPORAL_DIM = 1

_NC = 2
_NS = 16
_NW = _NC * _NS
_NBUF = 8
_HSPLIT = 4


def _subsample_indices(T, t):
    w = np.arange(t - 1, dtype=np.float32) / np.float32(t - 1)
    body = np.float32(0.0) * (np.float32(1.0) - w) + np.float32(T - 1) * w
    vals = np.concatenate([body, np.asarray([T - 1], np.float32)])
    vals = np.clip(vals, 0, T - 1)
    return vals.astype(np.int32)


def _check_static_indices(T, t16, t32):
    idx16 = [int(v) for v in _subsample_indices(T, t16)]
    idx32 = [int(v) for v in _subsample_indices(T, t32)]
    assert idx16 == [(j * (T - 1)) // (t16 - 1) for j in range(t16)]
    assert idx32 == [(j * (T - 1)) // (t32 - 1) for j in range(t32)]
    s16 = set(idx16)
    for j32, t in enumerate(idx32):
        shared = (idx16[j32 // 2] == t)
        assert shared == (t in s16)
        if shared:
            assert idx16[j32 // 2] == t
    only16 = [j for j, t in enumerate(idx16) if t not in set(idx32)]
    assert only16 == [5, 6, 7, 8, 9]


def kernel(x):
    B, T, C, H, W = x.shape
    t16, t32 = _NUM_SAMPLES
    _check_static_indices(T, t16, t32)

    S = _HSPLIT
    HH = H // S
    xr = x.reshape(B * T * C * S, HH, W)
    Q16 = B * t16 * C * S
    Q32 = B * t32 * C * S
    pwA = Q32 // _NW
    nOnly = 5
    QB = B * nOnly * C * S
    pwB = QB // _NW

    mesh = plsc.VectorSubcoreMesh(core_axis_name="c", subcore_axis_name="s")

    @functools.partial(
        pl.kernel,
        mesh=mesh,
        out_type=[jax.ShapeDtypeStruct((Q16, HH, W), x.dtype),
                  jax.ShapeDtypeStruct((Q32, HH, W), x.dtype)],
        scratch_types=(
            [pltpu.VMEM((1, HH, W), x.dtype) for _ in range(_NBUF)]
            + [pltpu.SemaphoreType.DMA for _ in range(3 * _NBUF)]
        ),
    )
    def run(x_hbm, o16_hbm, o32_hbm, *scratch):
        bufs = scratch[:_NBUF]
        gsems = scratch[_NBUF:2 * _NBUF]
        ssems = scratch[2 * _NBUF:3 * _NBUF]
        s2sems = scratch[3 * _NBUF:4 * _NBUF]
        wid = lax.axis_index("s") * _NC + lax.axis_index("c")

        baseA = wid * pwA

        def a_decomp(q):
            r = q // S
            h = q % S
            b = r // (t32 * C)
            rem = r % (t32 * C)
            j32 = rem // C
            c = rem % C
            return b, j32, c, h

        def a_shared(q):
            _, j32, _, _ = a_decomp(q)
            return (21 * (j32 // 2)) // 5 == ((T - 1) * j32) // (t32 - 1)

        def a_gather(q, slot):
            b, j32, c, h = a_decomp(q)
            tsrc = ((T - 1) * j32) // (t32 - 1)
            src = ((b * T + tsrc) * C + c) * S + h
            return pltpu.make_async_copy(
                x_hbm.at[pl.ds(src, 1)], bufs[slot], gsems[slot])

        def a_scatter(q, slot):
            return pltpu.make_async_copy(
                bufs[slot], o32_hbm.at[pl.ds(q, 1)], ssems[slot])

        def a_scatter16(q, slot):
            b, j32, c, h = a_decomp(q)
            dst16 = ((b * t16 + j32 // 2) * C + c) * S + h
            return pltpu.make_async_copy(
                bufs[slot], o16_hbm.at[pl.ds(dst16, 1)], s2sems[slot])

        niter = pwA // _NBUF
        for slot in range(_NBUF):
            a_gather(baseA + slot, slot).start()

        def bodyA(i, carry):
            q0 = baseA + i * _NBUF
            for slot in range(_NBUF):
                q = q0 + slot
                a_gather(q, slot).wait()
                a_scatter(q, slot).start()
                pl.when(a_shared(q))(lambda q=q, slot=slot:
                                     a_scatter16(q, slot).start())
            for slot in range(_NBUF):
                q = q0 + slot
                a_scatter(q, slot).wait()
                pl.when(a_shared(q))(lambda q=q, slot=slot:
                                     a_scatter16(q, slot).wait())
                a_gather(q + _NBUF, slot).start()
            return carry

        lax.fori_loop(0, niter - 1, bodyA, 0)
        qL = baseA + (niter - 1) * _NBUF
        for slot in range(_NBUF):
            q = qL + slot
            a_gather(q, slot).wait()
            a_scatter(q, slot).start()
            pl.when(a_shared(q))(lambda q=q, slot=slot:
                                 a_scatter16(q, slot).start())
        for slot in range(_NBUF):
            q = qL + slot
            a_scatter(q, slot).wait()
            pl.when(a_shared(q))(lambda q=q, slot=slot:
                                 a_scatter16(q, slot).wait())

        baseB = wid * pwB

        def b_copies(k, slot):
            e = baseB + k
            b = e // (nOnly * C * S)
            rem = e % (nOnly * C * S)
            m = rem // (C * S)
            c = (rem % (C * S)) // S
            h = rem % S
            j16 = nOnly + m
            tsrc = (21 * j16) // 5
            src = ((b * T + tsrc) * C + c) * S + h
            dst = ((b * t16 + j16) * C + c) * S + h
            g = pltpu.make_async_copy(
                x_hbm.at[pl.ds(src, 1)], bufs[slot], gsems[slot])
            s = pltpu.make_async_copy(
                bufs[slot], o16_hbm.at[pl.ds(dst, 1)], ssems[slot])
            return g, s

        for k in range(min(_NBUF, pwB)):
            b_copies(k, k)[0].start()
        pending = {}
        for k in range(pwB):
            slot = k % _NBUF
            g, s = b_copies(k, slot)
            g.wait()
            s.start()
            pending[slot] = k
            nxt = k + _NBUF
            if nxt < pwB:
                s.wait()
                del pending[slot]
                b_copies(nxt, slot)[0].start()
        for slot, k in pending.items():
            b_copies(k, slot)[1].wait()

    o16, o32 = run(xr)
    return (o16.reshape(B, t16, C, H, W), o32.reshape(B, t32, C, H, W))


# speedup vs baseline: 41.3042x; 1.0127x over previous
import functools

import numpy as np
import jax
import jax.numpy as jnp
from jax import lax
from jax.experimental import pallas as pl
from jax.experimental.pallas import tpu as pltpu
from jax.experimental.pallas import tpu_sc as plsc

_NUM_SAMPLES = (16, 32)
_TEMPORAL_DIM = 1

_NC = 2
_NS = 16
_NW = _NC * _NS
_NBUF = 8
_HSPLIT = 4


def _subsample_indices(T, t):
    w = np.arange(t - 1, dtype=np.float32) / np.float32(t - 1)
    body = np.float32(0.0) * (np.float32(1.0) - w) + np.float32(T - 1) * w
    vals = np.concatenate([body, np.asarray([T - 1], np.float32)])
    vals = np.clip(vals, 0, T - 1)
    return vals.astype(np.int32)


def _check_static_indices(T, t16, t32):
    idx16 = [int(v) for v in _subsample_indices(T, t16)]
    idx32 = [int(v) for v in _subsample_indices(T, t32)]
    assert idx16 == [(j * (T - 1)) // (t16 - 1) for j in range(t16)]
    assert idx32 == [(j * (T - 1)) // (t32 - 1) for j in range(t32)]
    s16 = set(idx16)
    for j32, t in enumerate(idx32):
        shared = (idx16[j32 // 2] == t)
        assert shared == (t in s16)
        if shared:
            assert idx16[j32 // 2] == t
    only16 = [j for j, t in enumerate(idx16) if t not in set(idx32)]
    assert only16 == [5, 6, 7, 8, 9]


def kernel(x):
    B, T, C, H, W = x.shape
    t16, t32 = _NUM_SAMPLES
    _check_static_indices(T, t16, t32)

    S = _HSPLIT
    HH = H // S
    xr = x.reshape(B * T * C * S, HH, W)
    Q16 = B * t16 * C * S
    Q32 = B * t32 * C * S
    pwA = Q32 // _NW
    nOnly = 5
    QB = B * nOnly * C * S
    pwB = QB // _NW

    mesh = plsc.VectorSubcoreMesh(core_axis_name="c", subcore_axis_name="s")

    @functools.partial(
        pl.kernel,
        mesh=mesh,
        out_type=[jax.ShapeDtypeStruct((Q16, HH, W), x.dtype),
                  jax.ShapeDtypeStruct((Q32, HH, W), x.dtype)],
        scratch_types=(
            [pltpu.VMEM((1, HH, W), x.dtype) for _ in range(_NBUF)]
            + [pltpu.SemaphoreType.DMA for _ in range(3 * _NBUF)]
        ),
    )
    def run(x_hbm, o16_hbm, o32_hbm, *scratch):
        bufs = scratch[:_NBUF]
        gsems = scratch[_NBUF:2 * _NBUF]
        ssems = scratch[2 * _NBUF:3 * _NBUF]
        s2sems = scratch[3 * _NBUF:4 * _NBUF]
        wid = lax.axis_index("s") * _NC + lax.axis_index("c")

        def a_chunk(k):
            return wid + _NW * k

        def a_decomp(q):
            r = q // S
            h = q % S
            b = r // (t32 * C)
            rem = r % (t32 * C)
            j32 = rem // C
            c = rem % C
            return b, j32, c, h

        def a_shared(q):
            _, j32, _, _ = a_decomp(q)
            return (21 * (j32 // 2)) // 5 == ((T - 1) * j32) // (t32 - 1)

        def a_gather(q, slot):
            b, j32, c, h = a_decomp(q)
            tsrc = ((T - 1) * j32) // (t32 - 1)
            src = ((b * T + tsrc) * C + c) * S + h
            return pltpu.make_async_copy(
                x_hbm.at[pl.ds(src, 1)], bufs[slot], gsems[slot])

        def a_scatter(q, slot):
            return pltpu.make_async_copy(
                bufs[slot], o32_hbm.at[pl.ds(q, 1)], ssems[slot])

        def a_scatter16(q, slot):
            b, j32, c, h = a_decomp(q)
            dst16 = ((b * t16 + j32 // 2) * C + c) * S + h
            return pltpu.make_async_copy(
                bufs[slot], o16_hbm.at[pl.ds(dst16, 1)], s2sems[slot])

        niter = pwA // _NBUF
        for slot in range(_NBUF):
            a_gather(a_chunk(slot), slot).start()

        def bodyA(i, carry):
            k0 = i * _NBUF
            for slot in range(_NBUF):
                q = a_chunk(k0 + slot)
                a_gather(q, slot).wait()
                a_scatter(q, slot).start()
                pl.when(a_shared(q))(lambda q=q, slot=slot:
                                     a_scatter16(q, slot).start())
            for slot in range(_NBUF):
                q = a_chunk(k0 + slot)
                a_scatter(q, slot).wait()
                pl.when(a_shared(q))(lambda q=q, slot=slot:
                                     a_scatter16(q, slot).wait())
                a_gather(a_chunk(k0 + _NBUF + slot), slot).start()
            return carry

        lax.fori_loop(0, niter - 1, bodyA, 0)
        kL = (niter - 1) * _NBUF
        for slot in range(_NBUF):
            q = a_chunk(kL + slot)
            a_gather(q, slot).wait()
            a_scatter(q, slot).start()
            pl.when(a_shared(q))(lambda q=q, slot=slot:
                                 a_scatter16(q, slot).start())
        for slot in range(_NBUF):
            q = a_chunk(kL + slot)
            a_scatter(q, slot).wait()
            pl.when(a_shared(q))(lambda q=q, slot=slot:
                                 a_scatter16(q, slot).wait())

        baseB = wid * pwB

        def b_copies(k, slot):
            e = baseB + k
            b = e // (nOnly * C * S)
            rem = e % (nOnly * C * S)
            m = rem // (C * S)
            c = (rem % (C * S)) // S
            h = rem % S
            j16 = nOnly + m
            tsrc = (21 * j16) // 5
            src = ((b * T + tsrc) * C + c) * S + h
            dst = ((b * t16 + j16) * C + c) * S + h
            g = pltpu.make_async_copy(
                x_hbm.at[pl.ds(src, 1)], bufs[slot], gsems[slot])
            s = pltpu.make_async_copy(
                bufs[slot], o16_hbm.at[pl.ds(dst, 1)], ssems[slot])
            return g, s

        for k in range(min(_NBUF, pwB)):
            b_copies(k, k)[0].start()
        pending = {}
        for k in range(pwB):
            slot = k % _NBUF
            g, s = b_copies(k, slot)
            g.wait()
            s.start()
            pending[slot] = k
            nxt = k + _NBUF
            if nxt < pwB:
                s.wait()
                del pending[slot]
                b_copies(nxt, slot)[0].start()
        for slot, k in pending.items():
            b_copies(k, slot)[1].wait()

    o16, o32 = run(xr)
    return (o16.reshape(B, t16, C, H, W), o32.reshape(B, t32, C, H, W))
